# baseline (speedup 1.0000x reference)
import functools

import jax
import jax.numpy as jnp
from jax import lax
from jax.experimental import pallas as pl
from jax.experimental.pallas import tpu as pltpu
from jax.experimental.pallas import tpu_sc as plsc

N = 100000
B = 16384
D = 128
TB = 2 * B
NW = 32
RPW = 3128
GPW = TB // NW
GCH = 256
CCH = 136
CTL = 40
ICH = 4096
MAXU = 3200
SCH = 128

_mesh = plsc.VectorSubcoreMesh(core_axis_name="c", subcore_axis_name="s")


def _wid():
    return lax.axis_index("s") * 2 + lax.axis_index("c")


@functools.partial(
    pl.kernel,
    out_type=jax.ShapeDtypeStruct((TB, D), jnp.float32),
    mesh=_mesh,
    compiler_params=pltpu.CompilerParams(needs_layout_passes=False),
    scratch_types=[
        pltpu.VMEM((GCH,), jnp.int32),
        pltpu.VMEM((GCH,), jnp.int32),
        pltpu.VMEM((GCH, D), jnp.float32),
        pltpu.VMEM((GCH, D), jnp.float32),
        [pltpu.SemaphoreType.DMA] * 2,
        [pltpu.SemaphoreType.DMA] * 2,
    ],
)
def _gather(ids_hbm, table_hbm, out_hbm, idx0, idx1, rows0, rows1,
            gsem, wsem):
    base = _wid() * GPW
    nc = GPW // GCH
    idx = [idx0, idx1]
    rows = [rows0, rows1]
    for c in range(nc):
        b = c % 2
        off = pl.multiple_of(base + c * GCH, GCH)
        if c >= 2:
            pltpu.make_async_copy(
                rows[b],
                out_hbm.at[pl.ds(pl.multiple_of(off - 2 * GCH, GCH), GCH)],
                wsem[b]).wait()
        pltpu.sync_copy(ids_hbm.at[pl.ds(off, GCH)], idx[b])
        pltpu.async_copy(table_hbm.at[idx[b]], rows[b], gsem[b]).wait()
        pltpu.async_copy(rows[b], out_hbm.at[pl.ds(off, GCH)], wsem[b])
    for c in range(nc - 2, nc):
        b = c % 2
        off = pl.multiple_of(base + c * GCH, GCH)
        pltpu.make_async_copy(rows[b], out_hbm.at[pl.ds(off, GCH)],
                              wsem[b]).wait()


def _mlp_body(x_ref, g_ref, wn_ref, bn_ref, wd_ref, bd_ref, o_ref):
    x = x_ref[...]
    g = g_ref[...]
    shift = lax.dot_general(
        x, wn_ref[...], (((1,), (1,)), ((), ())),
        precision=lax.Precision.HIGHEST,
        preferred_element_type=jnp.float32) + bn_ref[...]
    h = g + shift
    o_ref[...] = lax.dot_general(
        h, wd_ref[...], (((1,), (1,)), ((), ())),
        precision=lax.Precision.HIGHEST,
        preferred_element_type=jnp.float32) + bd_ref[...] + g


def _mlp(x, g, wn, bn, wd, bd):
    bm = 2048
    grid = (TB // bm,)
    return pl.pallas_call(
        _mlp_body,
        grid=grid,
        in_specs=[
            pl.BlockSpec((bm, D), lambda i: (i, 0)),
            pl.BlockSpec((bm, D), lambda i: (i, 0)),
            pl.BlockSpec((D, D), lambda i: (0, 0)),
            pl.BlockSpec((1, D), lambda i: (0, 0)),
            pl.BlockSpec((D, D), lambda i: (0, 0)),
            pl.BlockSpec((1, D), lambda i: (0, 0)),
        ],
        out_specs=pl.BlockSpec((bm, D), lambda i: (i, 0)),
        out_shape=jax.ShapeDtypeStruct((TB, D), jnp.float32),
    )(x, g, wn, bn, wd, bd)


@functools.partial(
    pl.kernel,
    out_type=jax.ShapeDtypeStruct((N, D), jnp.float32),
    mesh=_mesh,
    compiler_params=pltpu.CompilerParams(needs_layout_passes=False),
    scratch_types=[
        pltpu.VMEM((ICH,), jnp.int32),
        pltpu.VMEM((16,), jnp.int32),
        pltpu.VMEM((MAXU,), jnp.int32),
        pltpu.VMEM((MAXU,), jnp.int32),
        pltpu.VMEM((MAXU,), jnp.int32),
        pltpu.VMEM((SCH,), jnp.int32),
        pltpu.VMEM((SCH,), jnp.int32),
        pltpu.VMEM((SCH, D), jnp.float32),
        pltpu.SemaphoreType.DMA,
        pltpu.SemaphoreType.DMA,
        pltpu.SemaphoreType.DMA,
    ],
)
def _scatter(prev_hbm, ids_hbm, emb_hbm, out_hbm, idsv, sidv,
             winner, upos, uid, posw, idw, rowb, sem, sem2, csem):
    w = _wid()
    lo = w * RPW
    lane = lax.iota(jnp.int32, 16)
    last = w == NW - 1
    lo8 = pl.multiple_of(lo, 8)

    @pl.when(jnp.logical_not(last))
    def _():
        pltpu.async_copy(prev_hbm.at[pl.ds(lo8, RPW)],
                         out_hbm.at[pl.ds(lo8, RPW)], csem)

    @pl.when(last)
    def _():
        pltpu.async_copy(prev_hbm.at[pl.ds(lo8, N - (NW - 1) * RPW)],
                         out_hbm.at[pl.ds(lo8, N - (NW - 1) * RPW)], csem)

    def init_body(j, _):
        winner[pl.ds(j * 16, 16)] = jnp.full((16,), -1, jnp.int32)
        return 0
    lax.fori_loop(0, MAXU // 16, init_body, 0)

    def scan_chunk(ch, _):
        choff = pl.multiple_of(ch * ICH, ICH)
        pltpu.sync_copy(ids_hbm.at[pl.ds(choff, ICH)], idsv)

        def scan_vreg(i, _):
            idv = idsv[pl.ds(i * 16, 16)]
            inr = (idv >= lo) & (idv < lo + RPW)
            nactive = jnp.sum(inr.astype(jnp.int32))

            @pl.when(nactive > 0)
            def _():
                key = (idv << 4) | lane
                pos = choff + i * 16 + lane
                sk, spos = plsc.sort_key_val(key, pos, descending=True)
                sid = sk >> 4
                sidv[...] = sid
                prev_s = plsc.load_gather(sidv, [jnp.maximum(lane - 1, 0)])
                run_start = (sid != prev_s) | (lane == 0)
                slocal = sid - lo
                m = run_start & (slocal >= 0) & (slocal < RPW)
                cur = plsc.load_gather(winner, [slocal], mask=m)
                plsc.store_scatter(winner, [slocal],
                                   jnp.maximum(cur, spos), mask=m)
            return 0
        lax.fori_loop(0, ICH // 16, scan_vreg, 0)
        return 0
    lax.fori_loop(0, TB // ICH, scan_chunk, 0)

    def compact_body(j, off):
        wv = winner[pl.ds(j * 16, 16)]
        m = wv >= 0
        tgt = off + plsc.cumsum(m.astype(jnp.int32)) - 1
        plsc.store_scatter(upos, [tgt], wv, mask=m)
        plsc.store_scatter(uid, [tgt], lo + j * 16 + lane, mask=m)
        return off + jnp.sum(m.astype(jnp.int32))
    cnt = lax.fori_loop(0, MAXU // 16, compact_body, jnp.int32(0))

    cnt_pad = ((cnt + SCH - 1) // SCH) * SCH

    @pl.when(cnt > 0)
    def _():
        zero16 = jnp.zeros((16,), jnp.int32)
        p0 = plsc.load_gather(upos, [zero16])
        i0 = plsc.load_gather(uid, [zero16])

        def fill_body(j, _):
            vv = j * 16 + lane
            m = (vv >= cnt) & (vv < cnt_pad)
            upos[pl.ds(j * 16, 16)] = jnp.where(m, p0, upos[pl.ds(j * 16, 16)])
            uid[pl.ds(j * 16, 16)] = jnp.where(m, i0, uid[pl.ds(j * 16, 16)])
            return 0
        lax.fori_loop(cnt // 16, cnt_pad // 16, fill_body, 0)

    @pl.when(jnp.logical_not(last))
    def _():
        pltpu.make_async_copy(prev_hbm.at[pl.ds(lo8, RPW)],
                              out_hbm.at[pl.ds(lo8, RPW)], csem).wait()

    @pl.when(last)
    def _():
        pltpu.make_async_copy(
            prev_hbm.at[pl.ds(lo8, N - (NW - 1) * RPW)],
            out_hbm.at[pl.ds(lo8, N - (NW - 1) * RPW)], csem).wait()

    nch = cnt_pad // SCH

    def scat_body(c, _):
        @pl.when(c < nch)
        def _():
            for kk in range(SCH // 16):
                posw[pl.ds(kk * 16, 16)] = upos[pl.ds(c * SCH + kk * 16, 16)]
                idw[pl.ds(kk * 16, 16)] = uid[pl.ds(c * SCH + kk * 16, 16)]
            pltpu.async_copy(emb_hbm.at[posw], rowb, sem).wait()
            pltpu.async_copy(rowb, out_hbm.at[idw], sem2).wait()
        return 0
    lax.fori_loop(0, MAXU // SCH, scat_body, 0)


def kernel(src_node_ids, dst_node_ids, previous_embedding,
           batch_src_neighbor_embedding, batch_dst_neighbor_embedding,
           W_neighbor, b_neighbor, W_node, b_node):
    ids = jnp.concatenate([src_node_ids, dst_node_ids])
    x = jnp.concatenate([batch_src_neighbor_embedding,
                         batch_dst_neighbor_embedding], axis=0)
    gathered = _gather(ids, previous_embedding)
    emb = _mlp(x, gathered, W_neighbor, b_neighbor.reshape(1, D),
               W_node, b_node.reshape(1, D))
    return _scatter(previous_embedding, ids, emb)


# speedup vs baseline: 2.2259x; 2.2259x over previous
import functools

import jax
import jax.numpy as jnp
from jax import lax
from jax.experimental import pallas as pl
from jax.experimental.pallas import tpu as pltpu
from jax.experimental.pallas import tpu_sc as plsc

N = 100000
B = 16384
D = 128
TB = 2 * B
NW = 32
RPW = 3128
GPW = TB // NW
GCH = 256
CCH = 136
CTL = 40
ICH = 4096
MAXU = 3200
SCH = 128

_mesh = plsc.VectorSubcoreMesh(core_axis_name="c", subcore_axis_name="s")


def _wid():
    return lax.axis_index("s") * 2 + lax.axis_index("c")


@functools.partial(
    pl.kernel,
    out_type=(
        jax.ShapeDtypeStruct((TB, D), jnp.float32),
        jax.ShapeDtypeStruct((TB, D), jnp.float32),
        jax.ShapeDtypeStruct((TB,), jnp.int32),
    ),
    mesh=_mesh,
    compiler_params=pltpu.CompilerParams(needs_layout_passes=False),
    scratch_types=[
        pltpu.VMEM((GCH,), jnp.int32),
        pltpu.VMEM((GCH,), jnp.int32),
        pltpu.VMEM((GCH, D), jnp.float32),
        pltpu.VMEM((GCH, D), jnp.float32),
        [pltpu.SemaphoreType.DMA] * 2,
        [pltpu.SemaphoreType.DMA] * 2,
        pltpu.SemaphoreType.DMA,
        pltpu.SemaphoreType.DMA,
    ],
)
def _gather(src_hbm, dst_hbm, table_hbm, bsrc_hbm, bdst_hbm,
            out_hbm, xcat_hbm, idcat_hbm,
            idx0, idx1, rows0, rows1, gsem, wsem, xsem, isem):
    w = _wid()
    base = pl.multiple_of(w * GPW, GPW)
    loc = pl.multiple_of((w % (NW // 2)) * GPW, GPW)
    nc = GPW // GCH
    idx = [idx0, idx1]
    rows = [rows0, rows1]

    def run(ids_hbm, x_hbm):
        pltpu.async_copy(x_hbm.at[pl.ds(loc, GPW)],
                         xcat_hbm.at[pl.ds(base, GPW)], xsem)
        pltpu.async_copy(ids_hbm.at[pl.ds(loc, GPW)],
                         idcat_hbm.at[pl.ds(base, GPW)], isem)
        for c in range(nc):
            b = c % 2
            off = pl.multiple_of(base + c * GCH, GCH)
            src_off = pl.multiple_of(loc + c * GCH, GCH)
            if c >= 2:
                pltpu.make_async_copy(
                    rows[b],
                    out_hbm.at[pl.ds(pl.multiple_of(off - 2 * GCH, GCH),
                                     GCH)],
                    wsem[b]).wait()
            pltpu.sync_copy(ids_hbm.at[pl.ds(src_off, GCH)], idx[b])
            pltpu.async_copy(table_hbm.at[idx[b]], rows[b], gsem[b]).wait()
            pltpu.async_copy(rows[b], out_hbm.at[pl.ds(off, GCH)], wsem[b])
        for c in range(nc - 2, nc):
            b = c % 2
            off = pl.multiple_of(base + c * GCH, GCH)
            pltpu.make_async_copy(rows[b], out_hbm.at[pl.ds(off, GCH)],
                                  wsem[b]).wait()
        pltpu.make_async_copy(x_hbm.at[pl.ds(loc, GPW)],
                              xcat_hbm.at[pl.ds(base, GPW)], xsem).wait()
        pltpu.make_async_copy(ids_hbm.at[pl.ds(loc, GPW)],
                              idcat_hbm.at[pl.ds(base, GPW)], isem).wait()

    @pl.when(w < NW // 2)
    def _():
        run(src_hbm, bsrc_hbm)

    @pl.when(w >= NW // 2)
    def _():
        run(dst_hbm, bdst_hbm)


def _mlp_body(x_ref, g_ref, wn_ref, bn_ref, wd_ref, bd_ref, o_ref):
    x = x_ref[...]
    g = g_ref[...]
    shift = lax.dot_general(
        x, wn_ref[...], (((1,), (1,)), ((), ())),
        precision=lax.Precision.HIGHEST,
        preferred_element_type=jnp.float32) + bn_ref[...]
    h = g + shift
    o_ref[...] = lax.dot_general(
        h, wd_ref[...], (((1,), (1,)), ((), ())),
        precision=lax.Precision.HIGHEST,
        preferred_element_type=jnp.float32) + bd_ref[...] + g


def _mlp(x, g, wn, bn, wd, bd):
    bm = 2048
    grid = (TB // bm,)
    return pl.pallas_call(
        _mlp_body,
        grid=grid,
        in_specs=[
            pl.BlockSpec((bm, D), lambda i: (i, 0)),
            pl.BlockSpec((bm, D), lambda i: (i, 0)),
            pl.BlockSpec((D, D), lambda i: (0, 0)),
            pl.BlockSpec((1, D), lambda i: (0, 0)),
            pl.BlockSpec((D, D), lambda i: (0, 0)),
            pl.BlockSpec((1, D), lambda i: (0, 0)),
        ],
        out_specs=pl.BlockSpec((bm, D), lambda i: (i, 0)),
        out_shape=jax.ShapeDtypeStruct((TB, D), jnp.float32),
    )(x, g, wn, bn, wd, bd)


@functools.partial(
    pl.kernel,
    out_type=jax.ShapeDtypeStruct((N, D), jnp.float32),
    mesh=_mesh,
    compiler_params=pltpu.CompilerParams(needs_layout_passes=False),
    scratch_types=[
        pltpu.VMEM((CCH, D), jnp.float32),
        pltpu.VMEM((CCH, D), jnp.float32),
        pltpu.VMEM((CTL, D), jnp.float32),
        [pltpu.SemaphoreType.DMA] * 2,
        [pltpu.SemaphoreType.DMA] * 2,
        pltpu.VMEM((ICH,), jnp.int32),
        pltpu.VMEM((16,), jnp.int32),
        pltpu.VMEM((MAXU,), jnp.int32),
        pltpu.VMEM((MAXU,), jnp.int32),
        pltpu.VMEM((MAXU,), jnp.int32),
        pltpu.VMEM((SCH,), jnp.int32),
        pltpu.VMEM((SCH,), jnp.int32),
        pltpu.VMEM((SCH, D), jnp.float32),
        pltpu.SemaphoreType.DMA,
        pltpu.SemaphoreType.DMA,
    ],
)
def _scatter(prev_hbm, ids_hbm, emb_hbm, out_hbm, cb0, cb1, ctbuf,
             rsem, wsem, idsv, sidv,
             winner, upos, uid, posw, idw, rowb, sem, sem2):
    w = _wid()
    lo = w * RPW
    lane = lax.iota(jnp.int32, 16)
    last = w == NW - 1

    cb = [cb0, cb1]
    nfull = jnp.where(last, 22, 23)

    for c in range(23):
        b = c % 2

        @pl.when(c < nfull)
        def _(c=c, b=b):
            off = pl.multiple_of(lo + c * CCH, 8)
            if c >= 2:
                pltpu.make_async_copy(
                    cb[b],
                    out_hbm.at[pl.ds(pl.multiple_of(lo + (c - 2) * CCH, 8),
                                     CCH)],
                    wsem[b]).wait()
            pltpu.async_copy(prev_hbm.at[pl.ds(off, CCH)], cb[b], rsem[b])
            pltpu.make_async_copy(prev_hbm.at[pl.ds(off, CCH)], cb[b],
                                  rsem[b]).wait()
            pltpu.async_copy(cb[b], out_hbm.at[pl.ds(off, CCH)], wsem[b])

    @pl.when(last)
    def _():
        off = pl.multiple_of(lo + 22 * CCH, 8)
        pltpu.sync_copy(prev_hbm.at[pl.ds(off, CTL)], ctbuf)
        pltpu.async_copy(ctbuf, out_hbm.at[pl.ds(off, CTL)], rsem[0])
        pltpu.make_async_copy(ctbuf, out_hbm.at[pl.ds(off, CTL)],
                              rsem[0]).wait()

    for b in range(2):
        pltpu.make_async_copy(cb[b], out_hbm.at[pl.ds(lo, CCH)],
                              wsem[b]).wait()

    def init_body(j, _):
        winner[pl.ds(j * 16, 16)] = jnp.full((16,), -1, jnp.int32)
        return 0
    lax.fori_loop(0, MAXU // 16, init_body, 0)

    def scan_chunk(ch, _):
        choff = pl.multiple_of(ch * ICH, ICH)
        pltpu.sync_copy(ids_hbm.at[pl.ds(choff, ICH)], idsv)

        def scan_vreg(i, _):
            idv = idsv[pl.ds(i * 16, 16)]
            inr = (idv >= lo) & (idv < lo + RPW)
            nactive = jnp.sum(inr.astype(jnp.int32))

            @pl.when(nactive > 0)
            def _():
                key = (idv << 4) | lane
                pos = choff + i * 16 + lane
                sk, spos = plsc.sort_key_val(key, pos, descending=True)
                sid = sk >> 4
                sidv[...] = sid
                prev_s = plsc.load_gather(sidv, [jnp.maximum(lane - 1, 0)])
                run_start = (sid != prev_s) | (lane == 0)
                slocal = sid - lo
                m = run_start & (slocal >= 0) & (slocal < RPW)
                cur = plsc.load_gather(winner, [slocal], mask=m)
                plsc.store_scatter(winner, [slocal],
                                   jnp.maximum(cur, spos), mask=m)
            return 0
        lax.fori_loop(0, ICH // 16, scan_vreg, 0)
        return 0
    lax.fori_loop(0, TB // ICH, scan_chunk, 0)

    def compact_body(j, off):
        wv = winner[pl.ds(j * 16, 16)]
        m = wv >= 0
        tgt = off + plsc.cumsum(m.astype(jnp.int32)) - 1
        plsc.store_scatter(upos, [tgt], wv, mask=m)
        plsc.store_scatter(uid, [tgt], lo + j * 16 + lane, mask=m)
        return off + jnp.sum(m.astype(jnp.int32))
    cnt = lax.fori_loop(0, MAXU // 16, compact_body, jnp.int32(0))

    cnt_pad = ((cnt + SCH - 1) // SCH) * SCH

    @pl.when(cnt > 0)
    def _():
        zero16 = jnp.zeros((16,), jnp.int32)
        p0 = plsc.load_gather(upos, [zero16])
        i0 = plsc.load_gather(uid, [zero16])

        def fill_body(j, _):
            vv = j * 16 + lane
            m = (vv >= cnt) & (vv < cnt_pad)
            upos[pl.ds(j * 16, 16)] = jnp.where(m, p0, upos[pl.ds(j * 16, 16)])
            uid[pl.ds(j * 16, 16)] = jnp.where(m, i0, uid[pl.ds(j * 16, 16)])
            return 0
        lax.fori_loop(cnt // 16, cnt_pad // 16, fill_body, 0)

    nch = cnt_pad // SCH

    def scat_body(c, _):
        @pl.when(c < nch)
        def _():
            for kk in range(SCH // 16):
                posw[pl.ds(kk * 16, 16)] = upos[pl.ds(c * SCH + kk * 16, 16)]
                idw[pl.ds(kk * 16, 16)] = uid[pl.ds(c * SCH + kk * 16, 16)]
            pltpu.async_copy(emb_hbm.at[posw], rowb, sem).wait()
            pltpu.async_copy(rowb, out_hbm.at[idw], sem2).wait()
        return 0
    lax.fori_loop(0, MAXU // SCH, scat_body, 0)


def kernel(src_node_ids, dst_node_ids, previous_embedding,
           batch_src_neighbor_embedding, batch_dst_neighbor_embedding,
           W_neighbor, b_neighbor, W_node, b_node):
    gathered, xcat, idcat = _gather(
        src_node_ids, dst_node_ids, previous_embedding,
        batch_src_neighbor_embedding, batch_dst_neighbor_embedding)
    emb = _mlp(xcat, gathered, W_neighbor, b_neighbor.reshape(1, D),
               W_node, b_node.reshape(1, D))
    return _scatter(previous_embedding, idcat, emb)


# speedup vs baseline: 8.1419x; 3.6578x over previous
import functools

import jax
import jax.numpy as jnp
from jax import lax
from jax.experimental import pallas as pl
from jax.experimental.pallas import tpu as pltpu
from jax.experimental.pallas import tpu_sc as plsc

N = 100000
B = 16384
D = 128
TB = 2 * B
NW = 32
RPW = 3128
GPW = TB // NW
GCH = 256
CCH = 136
CTL = 40
ICH = 4096
MAXU = 3200
SCH = 128

_mesh = plsc.VectorSubcoreMesh(core_axis_name="c", subcore_axis_name="s")


def _wid():
    return lax.axis_index("s") * 2 + lax.axis_index("c")


@functools.partial(
    pl.kernel,
    out_type=jax.ShapeDtypeStruct((TB, D), jnp.float32),
    mesh=_mesh,
    compiler_params=pltpu.CompilerParams(needs_layout_passes=False),
    scratch_types=[
        pltpu.VMEM((GCH,), jnp.int32),
        pltpu.VMEM((GCH,), jnp.int32),
        pltpu.VMEM((GCH, D), jnp.float32),
        pltpu.VMEM((GCH, D), jnp.float32),
        [pltpu.SemaphoreType.DMA] * 2,
        [pltpu.SemaphoreType.DMA] * 2,
    ],
)
def _gather(ids_hbm, table_hbm, out_hbm, idx0, idx1, rows0, rows1,
            gsem, wsem):
    base = pl.multiple_of(_wid() * GPW, GPW)
    nc = GPW // GCH
    idx = [idx0, idx1]
    rows = [rows0, rows1]
    for c in range(nc):
        b = c % 2
        off = pl.multiple_of(base + c * GCH, GCH)
        if c >= 2:
            pltpu.make_async_copy(
                rows[b],
                out_hbm.at[pl.ds(pl.multiple_of(off - 2 * GCH, GCH), GCH)],
                wsem[b]).wait()
        pltpu.sync_copy(ids_hbm.at[pl.ds(off, GCH)], idx[b])
        pltpu.async_copy(table_hbm.at[idx[b]], rows[b], gsem[b]).wait()
        pltpu.async_copy(rows[b], out_hbm.at[pl.ds(off, GCH)], wsem[b])
    for c in range(nc - 2, nc):
        b = c % 2
        off = pl.multiple_of(base + c * GCH, GCH)
        pltpu.make_async_copy(rows[b], out_hbm.at[pl.ds(off, GCH)],
                              wsem[b]).wait()


def _mlp_body(x_ref, g_ref, wn_ref, bn_ref, wd_ref, bd_ref, o_ref):
    x = x_ref[...]
    g = g_ref[...]
    shift = lax.dot_general(
        x, wn_ref[...], (((1,), (1,)), ((), ())),
        precision=lax.Precision.HIGHEST,
        preferred_element_type=jnp.float32) + bn_ref[...]
    h = g + shift
    o_ref[...] = lax.dot_general(
        h, wd_ref[...], (((1,), (1,)), ((), ())),
        precision=lax.Precision.HIGHEST,
        preferred_element_type=jnp.float32) + bd_ref[...] + g


def _mlp(x, g, wn, bn, wd, bd):
    bm = 2048
    grid = (TB // bm,)
    return pl.pallas_call(
        _mlp_body,
        grid=grid,
        in_specs=[
            pl.BlockSpec((bm, D), lambda i: (i, 0)),
            pl.BlockSpec((bm, D), lambda i: (i, 0)),
            pl.BlockSpec((D, D), lambda i: (0, 0)),
            pl.BlockSpec((1, D), lambda i: (0, 0)),
            pl.BlockSpec((D, D), lambda i: (0, 0)),
            pl.BlockSpec((1, D), lambda i: (0, 0)),
        ],
        out_specs=pl.BlockSpec((bm, D), lambda i: (i, 0)),
        out_shape=jax.ShapeDtypeStruct((TB, D), jnp.float32),
    )(x, g, wn, bn, wd, bd)


@functools.partial(
    pl.kernel,
    out_type=jax.ShapeDtypeStruct((N, D), jnp.float32),
    mesh=_mesh,
    compiler_params=pltpu.CompilerParams(needs_layout_passes=False),
    scratch_types=[
        pltpu.VMEM((CCH, D), jnp.float32),
        pltpu.VMEM((CCH, D), jnp.float32),
        pltpu.VMEM((CTL, D), jnp.float32),
        [pltpu.SemaphoreType.DMA] * 2,
        [pltpu.SemaphoreType.DMA] * 2,
        pltpu.SemaphoreType.DMA,
        pltpu.VMEM((TB,), jnp.int32),
        pltpu.VMEM((16,), jnp.int32),
        pltpu.VMEM((MAXU,), jnp.int32),
        pltpu.VMEM((MAXU,), jnp.int32),
        pltpu.VMEM((MAXU,), jnp.int32),
        pltpu.VMEM((SCH,), jnp.int32),
        pltpu.VMEM((SCH,), jnp.int32),
        pltpu.VMEM((SCH, D), jnp.float32),
        pltpu.SemaphoreType.DMA,
        pltpu.SemaphoreType.DMA,
    ],
)
def _scatter(prev_hbm, ids_hbm, emb_hbm, out_hbm, cb0, cb1, ctbuf,
             rsem, wsem, isem, idsv, sidv,
             winner, upos, uid, posw, idw, rowb, sem, sem2):
    w = _wid()
    lo = w * RPW
    lane = lax.iota(jnp.int32, 16)
    last = w == NW - 1

    pltpu.async_copy(ids_hbm, idsv, isem)

    def init_body(j, _):
        winner[pl.ds(j * 16, 16)] = jnp.full((16,), -1, jnp.int32)
        return 0
    lax.fori_loop(0, MAXU // 16, init_body, 0)

    pltpu.make_async_copy(ids_hbm, idsv, isem).wait()

    def scan_range(s, e):
        def scan_vreg(i, _):
            idv = idsv[pl.ds(i * 16, 16)]
            inr = (idv >= lo) & (idv < lo + RPW)
            nactive = plsc.all_reduce_population_count(inr)

            @pl.when(nactive[0] > 0)
            def _():
                key = (idv << 4) | lane
                pos = i * 16 + lane
                sk, spos = plsc.sort_key_val(key, pos, descending=True)
                sid = sk >> 4
                sidv[...] = sid
                prev_s = plsc.load_gather(sidv, [jnp.maximum(lane - 1, 0)])
                run_start = (sid != prev_s) | (lane == 0)
                slocal = sid - lo
                m = run_start & (slocal >= 0) & (slocal < RPW)
                cur = plsc.load_gather(winner, [slocal], mask=m)
                plsc.store_scatter(winner, [slocal],
                                   jnp.maximum(cur, spos), mask=m)
            return 0
        lax.fori_loop(s, e, scan_vreg, 0)

    cb = [cb0, cb1]
    nfull = jnp.where(last, 22, 23)
    NV = TB // 16
    VPS = NV // 23

    for c in range(23):
        b = c % 2

        @pl.when(c < nfull)
        def _(c=c, b=b):
            off = pl.multiple_of(lo + c * CCH, 8)
            if c >= 2:
                pltpu.make_async_copy(
                    cb[b],
                    out_hbm.at[pl.ds(pl.multiple_of(lo + (c - 2) * CCH, 8),
                                     CCH)],
                    wsem[b]).wait()
            pltpu.async_copy(prev_hbm.at[pl.ds(off, CCH)], cb[b], rsem[b])

        scan_range(c * VPS, (c + 1) * VPS if c < 22 else NV)

        @pl.when(c < nfull)
        def _(c=c, b=b):
            off = pl.multiple_of(lo + c * CCH, 8)
            pltpu.make_async_copy(prev_hbm.at[pl.ds(off, CCH)], cb[b],
                                  rsem[b]).wait()
            pltpu.async_copy(cb[b], out_hbm.at[pl.ds(off, CCH)], wsem[b])

    @pl.when(last)
    def _():
        off = pl.multiple_of(lo + 22 * CCH, 8)
        pltpu.sync_copy(prev_hbm.at[pl.ds(off, CTL)], ctbuf)
        pltpu.async_copy(ctbuf, out_hbm.at[pl.ds(off, CTL)], rsem[0])
        pltpu.make_async_copy(ctbuf, out_hbm.at[pl.ds(off, CTL)],
                              rsem[0]).wait()

    for b in range(2):
        pltpu.make_async_copy(cb[b], out_hbm.at[pl.ds(lo, CCH)],
                              wsem[b]).wait()

    def compact_body(j, off):
        wv = winner[pl.ds(j * 16, 16)]
        m = wv >= 0
        tgt = off + plsc.cumsum(m.astype(jnp.int32)) - 1
        plsc.store_scatter(upos, [tgt], wv, mask=m)
        plsc.store_scatter(uid, [tgt], lo + j * 16 + lane, mask=m)
        return off + jnp.sum(m.astype(jnp.int32))
    cnt = lax.fori_loop(0, MAXU // 16, compact_body, jnp.int32(0))

    cnt_pad = ((cnt + SCH - 1) // SCH) * SCH

    @pl.when(cnt > 0)
    def _():
        zero16 = jnp.zeros((16,), jnp.int32)
        p0 = plsc.load_gather(upos, [zero16])
        i0 = plsc.load_gather(uid, [zero16])

        def fill_body(j, _):
            vv = j * 16 + lane
            m = (vv >= cnt) & (vv < cnt_pad)
            upos[pl.ds(j * 16, 16)] = jnp.where(m, p0, upos[pl.ds(j * 16, 16)])
            uid[pl.ds(j * 16, 16)] = jnp.where(m, i0, uid[pl.ds(j * 16, 16)])
            return 0
        lax.fori_loop(cnt // 16, cnt_pad // 16, fill_body, 0)

    nch = cnt_pad // SCH

    def scat_body(c, _):
        @pl.when(c < nch)
        def _():
            for kk in range(SCH // 16):
                posw[pl.ds(kk * 16, 16)] = upos[pl.ds(c * SCH + kk * 16, 16)]
                idw[pl.ds(kk * 16, 16)] = uid[pl.ds(c * SCH + kk * 16, 16)]
            pltpu.async_copy(emb_hbm.at[posw], rowb, sem).wait()
            pltpu.async_copy(rowb, out_hbm.at[idw], sem2).wait()
        return 0
    lax.fori_loop(0, MAXU // SCH, scat_body, 0)


def kernel(src_node_ids, dst_node_ids, previous_embedding,
           batch_src_neighbor_embedding, batch_dst_neighbor_embedding,
           W_neighbor, b_neighbor, W_node, b_node):
    ids = jnp.concatenate([src_node_ids, dst_node_ids])
    x = jnp.concatenate([batch_src_neighbor_embedding,
                         batch_dst_neighbor_embedding], axis=0)
    gathered = _gather(ids, previous_embedding)
    emb = _mlp(x, gathered, W_neighbor, b_neighbor.reshape(1, D),
               W_node, b_node.reshape(1, D))
    return _scatter(previous_embedding, ids, emb)
